# baseline (speedup 1.0000x reference)
import functools

import jax
import jax.numpy as jnp
from jax import lax
from jax.experimental import pallas as pl
from jax.experimental.pallas import tpu as pltpu
from jax.experimental.pallas import tpu_sc as plsc

VOCAB = 100000
DIM = 64
PDIM = 128
BATCH = 4096
SEQ = 200
NUM_TOK = BATCH * SEQ
NUM_WORKERS = 32
ROWS_W = BATCH // NUM_WORKERS
PER_W = NUM_TOK // NUM_WORKERS
GHALF = 128
VCHUNK = 400
NVCHUNKS = VOCAB // VCHUNK
UNROLL = 8

_mesh = plsc.VectorSubcoreMesh(core_axis_name="c", subcore_axis_name="s")


def _positional_encoding(token_length, embedding_dim):
    pos = jnp.arange(token_length, dtype=jnp.float32)[:, None]
    i = jnp.arange(embedding_dim)[None, :]
    angle_rates = 1.0 / jnp.power(
        10000.0, (2 * (i // 2)).astype(jnp.float32) / embedding_dim)
    angles = pos * angle_rates
    return jnp.where(i % 2 == 0, jnp.sin(angles), jnp.cos(angles)).astype(jnp.float32)


_PREP_J = (NVCHUNKS + NUM_WORKERS - 1) // NUM_WORKERS


@functools.partial(
    pl.kernel,
    mesh=_mesh,
    out_type=jax.ShapeDtypeStruct((2 * VOCAB, PDIM), jnp.float32),
    scratch_types=[
        pltpu.VMEM((2, DIM), jnp.float32),
        pltpu.VMEM((VCHUNK // 2, PDIM), jnp.float32),
        pltpu.VMEM((VCHUNK, PDIM), jnp.float32),
        pltpu.VMEM((VCHUNK, PDIM), jnp.float32),
        pltpu.SemaphoreType.DMA,
    ],
    compiler_params=pltpu.CompilerParams(use_tc_tiling_on_sc=True),
)
def _prep_kernel(word2_hbm, type_hbm, vocab2_hbm, ty_v, wch, va, vb, wsem):
    wid = lax.axis_index("s") * 2 + lax.axis_index("c")
    pltpu.sync_copy(type_hbm, ty_v)

    def wait_write():
        dummy = vocab2_hbm.at[pl.ds(0, VCHUNK)]
        pltpu.make_async_copy(va, dummy, wsem).wait()
        pltpu.make_async_copy(vb, dummy, wsem).wait()

    def chunk_body(j, carry):
        ci = j * NUM_WORKERS + wid

        @pl.when(ci < NVCHUNKS)
        def _():
            r0 = pl.multiple_of(ci * (VCHUNK // 2), 8)
            pltpu.sync_copy(word2_hbm.at[pl.ds(r0, VCHUNK // 2)], wch)

            @pl.when(j >= 1)
            def _():
                wait_write()

            def row_body(i2, c2):
                for u in range(2):
                    for jj in range(DIM // 16):
                        w = wch[i2, pl.ds(u * DIM + jj * 16, 16)]
                        s = pl.ds(jj * 16, 16)
                        va[i2 * 2 + u, s] = (w + ty_v[0, s]) * 8.0
                        vb[i2 * 2 + u, s] = (w + ty_v[1, s]) * 8.0
                return c2

            lax.fori_loop(0, VCHUNK // 2, row_body, 0)
            pltpu.async_copy(
                va, vocab2_hbm.at[pl.ds(pl.multiple_of(ci * VCHUNK, 8),
                                        VCHUNK)], wsem)
            pltpu.async_copy(
                vb, vocab2_hbm.at[pl.ds(pl.multiple_of(VOCAB + ci * VCHUNK, 8),
                                        VCHUNK)], wsem)
        return carry

    lax.fori_loop(0, _PREP_J, chunk_body, 0)
    wait_write()


@functools.partial(
    pl.kernel,
    mesh=_mesh,
    out_type=jax.ShapeDtypeStruct((NUM_TOK, DIM), jnp.float32),
    scratch_types=[
        pltpu.VMEM((PER_W,), jnp.int32),
        pltpu.VMEM((SEQ, DIM), jnp.float32),
        pltpu.VMEM((SEQ, PDIM), jnp.float32),
        pltpu.VMEM((SEQ, PDIM), jnp.float32),
        pltpu.VMEM((SEQ, DIM), jnp.float32),
        pltpu.SemaphoreType.DMA,
        pltpu.SemaphoreType.DMA,
        pltpu.SemaphoreType.DMA,
        pltpu.SemaphoreType.DMA,
    ],
    compiler_params=pltpu.CompilerParams(use_tc_tiling_on_sc=True),
)
def _emb_kernel(idx_hbm, pe_hbm, vocab2_hbm, out_hbm,
                idx_v, pe_v, w0, w1, obuf, g0, g1, wr0, wr1):
    wid = lax.axis_index("s") * 2 + lax.axis_index("c")
    base = pl.multiple_of(wid * PER_W, PER_W)
    pltpu.sync_copy(idx_hbm.at[pl.ds(base, PER_W)], idx_v)
    pltpu.sync_copy(pe_hbm, pe_v)

    wbufs, gsems, wsems = (w0, w1), (g0, g1), (wr0, wr1)

    def issue_gather(it, b):
        off = pl.multiple_of(it * SEQ, 8)
        pltpu.async_copy(vocab2_hbm.at[idx_v.at[pl.ds(off, GHALF)]],
                         wbufs[b].at[pl.ds(0, GHALF)], gsems[b])
        pltpu.async_copy(vocab2_hbm.at[idx_v.at[pl.ds(off + GHALF, SEQ - GHALF)]],
                         wbufs[b].at[pl.ds(GHALF, SEQ - GHALF)], gsems[b])

    def wait_gather(b):
        pltpu.make_async_copy(vocab2_hbm.at[pl.ds(0, SEQ)], wbufs[b],
                              gsems[b]).wait()

    def issue_write(it, b):
        off = pl.multiple_of(it * SEQ, 8)
        pltpu.async_copy(obuf, out_hbm.at[pl.ds(base + off, SEQ)], wsems[b])

    def wait_write(b):
        pltpu.make_async_copy(obuf, out_hbm.at[pl.ds(0, SEQ)],
                              wsems[b]).wait()

    def compute(b):
        wbuf = wbufs[b]

        def row_body(r, c2):
            for u in range(UNROLL):
                i = r * UNROLL + u
                for j in range(DIM // 16):
                    s = pl.ds(j * 16, 16)
                    obuf[i, s] = wbuf[i, s] + pe_v[i, s]
            return c2

        lax.fori_loop(0, SEQ // UNROLL, row_body, 0)

    issue_gather(0, 0)

    def pair_body(gi, carry):
        for b in range(2):
            it = gi * 2 + b
            nb = 1 - b
            if b == 0:
                @pl.when(gi >= 1)
                def _():
                    wait_write(nb)
                issue_gather(it + 1, nb)
            else:
                wait_write(nb)

                @pl.when(gi + 1 < ROWS_W // 2)
                def _():
                    issue_gather(it + 1, nb)
            wait_gather(b)
            compute(b)
            issue_write(it, b)
        return carry

    lax.fori_loop(0, ROWS_W // 2, pair_body, 0)
    wait_write(1)


def kernel(tokens, token_types, word_table, type_table):
    idx = (tokens.astype(jnp.int32)
           + VOCAB * token_types.astype(jnp.int32)).reshape(NUM_TOK)
    pe = _positional_encoding(SEQ, DIM)
    word2 = word_table.reshape(VOCAB // 2, PDIM)
    vocab2 = _prep_kernel(word2, type_table)
    out = _emb_kernel(idx, pe, vocab2)
    return out.reshape(BATCH, SEQ, DIM)


# speedup vs baseline: 1.0446x; 1.0446x over previous
import functools

import jax
import jax.numpy as jnp
from jax import lax
from jax.experimental import pallas as pl
from jax.experimental.pallas import tpu as pltpu
from jax.experimental.pallas import tpu_sc as plsc

VOCAB = 100000
DIM = 64
PDIM = 128
BATCH = 4096
SEQ = 200
NUM_TOK = BATCH * SEQ
NUM_WORKERS = 32
ROWS_W = BATCH // NUM_WORKERS
PER_W = NUM_TOK // NUM_WORKERS
GHALF = 128
VCHUNK = 160
NVCHUNKS = VOCAB // VCHUNK
UNROLL = 8

_mesh = plsc.VectorSubcoreMesh(core_axis_name="c", subcore_axis_name="s")


def _positional_encoding(token_length, embedding_dim):
    pos = jnp.arange(token_length, dtype=jnp.float32)[:, None]
    i = jnp.arange(embedding_dim)[None, :]
    angle_rates = 1.0 / jnp.power(
        10000.0, (2 * (i // 2)).astype(jnp.float32) / embedding_dim)
    angles = pos * angle_rates
    return jnp.where(i % 2 == 0, jnp.sin(angles), jnp.cos(angles)).astype(jnp.float32)


_PREP_J = (NVCHUNKS + NUM_WORKERS - 1) // NUM_WORKERS


@functools.partial(
    pl.kernel,
    mesh=_mesh,
    out_type=jax.ShapeDtypeStruct((2 * VOCAB, PDIM), jnp.float32),
    scratch_types=[
        pltpu.VMEM((2, DIM), jnp.float32),
        pltpu.VMEM((VCHUNK // 2, PDIM), jnp.float32),
        pltpu.VMEM((VCHUNK // 2, PDIM), jnp.float32),
        pltpu.VMEM((VCHUNK, PDIM), jnp.float32),
        pltpu.VMEM((VCHUNK, PDIM), jnp.float32),
        pltpu.VMEM((VCHUNK, PDIM), jnp.float32),
        pltpu.VMEM((VCHUNK, PDIM), jnp.float32),
        pltpu.SemaphoreType.DMA,
        pltpu.SemaphoreType.DMA,
        pltpu.SemaphoreType.DMA,
        pltpu.SemaphoreType.DMA,
    ],
    compiler_params=pltpu.CompilerParams(use_tc_tiling_on_sc=True),
)
def _prep_kernel(word2_hbm, type_hbm, vocab2_hbm, ty_v,
                 wch0, wch1, va0, vb0, va1, vb1, rd0, rd1, wr0, wr1):
    wid = lax.axis_index("s") * 2 + lax.axis_index("c")
    pltpu.sync_copy(type_hbm, ty_v)

    wchs, vas, vbs = (wch0, wch1), (va0, va1), (vb0, vb1)
    rds, wrs = (rd0, rd1), (wr0, wr1)

    def issue_read(ci, b):
        r0 = pl.multiple_of(ci * (VCHUNK // 2), 8)
        pltpu.async_copy(word2_hbm.at[pl.ds(r0, VCHUNK // 2)], wchs[b], rds[b])

    def wait_read(b):
        pltpu.make_async_copy(word2_hbm.at[pl.ds(0, VCHUNK // 2)], wchs[b],
                              rds[b]).wait()

    def issue_write(ci, b):
        r0 = pl.multiple_of(ci * VCHUNK, 8)
        pltpu.async_copy(vas[b], vocab2_hbm.at[pl.ds(r0, VCHUNK)], wrs[b])
        pltpu.async_copy(
            vbs[b],
            vocab2_hbm.at[pl.ds(pl.multiple_of(VOCAB + ci * VCHUNK, 8),
                                VCHUNK)], wrs[b])

    def wait_write(b):
        dummy = vocab2_hbm.at[pl.ds(0, VCHUNK)]
        pltpu.make_async_copy(vas[b], dummy, wrs[b]).wait()
        pltpu.make_async_copy(vbs[b], dummy, wrs[b]).wait()

    def compute(b):
        wch, va, vb = wchs[b], vas[b], vbs[b]

        def row_body(i2, c2):
            for u in range(2):
                for jj in range(DIM // 16):
                    w = wch[i2, pl.ds(u * DIM + jj * 16, 16)]
                    s = pl.ds(jj * 16, 16)
                    va[i2 * 2 + u, s] = (w + ty_v[0, s]) * 8.0
                    vb[i2 * 2 + u, s] = (w + ty_v[1, s]) * 8.0
            return c2

        lax.fori_loop(0, VCHUNK // 2, row_body, 0)

    @pl.when(wid < NVCHUNKS)
    def _():
        issue_read(wid, 0)

    def pair_body(g, carry):
        for b in range(2):
            j = g * 2 + b
            ci = j * NUM_WORKERS + wid
            cnext = ci + NUM_WORKERS

            @pl.when(cnext < NVCHUNKS)
            def _():
                issue_read(cnext, 1 - b)

            @pl.when(ci < NVCHUNKS)
            def _():
                wait_read(b)

                @pl.when(g >= 1)
                def _():
                    wait_write(b)
                compute(b)
                issue_write(ci, b)
        return carry

    lax.fori_loop(0, _PREP_J // 2, pair_body, 0)
    for b in range(2):
        last_ci = (_PREP_J - 2 + b) * NUM_WORKERS + wid

        @pl.when(last_ci < NVCHUNKS)
        def _():
            wait_write(b)


@functools.partial(
    pl.kernel,
    mesh=_mesh,
    out_type=jax.ShapeDtypeStruct((NUM_TOK, DIM), jnp.float32),
    scratch_types=[
        pltpu.VMEM((PER_W,), jnp.int32),
        pltpu.VMEM((SEQ, DIM), jnp.float32),
        pltpu.VMEM((SEQ, PDIM), jnp.float32),
        pltpu.VMEM((SEQ, PDIM), jnp.float32),
        pltpu.VMEM((SEQ, DIM), jnp.float32),
        pltpu.SemaphoreType.DMA,
        pltpu.SemaphoreType.DMA,
        pltpu.SemaphoreType.DMA,
        pltpu.SemaphoreType.DMA,
    ],
    compiler_params=pltpu.CompilerParams(use_tc_tiling_on_sc=True),
)
def _emb_kernel(idx_hbm, pe_hbm, vocab2_hbm, out_hbm,
                idx_v, pe_v, w0, w1, obuf, g0, g1, wr0, wr1):
    wid = lax.axis_index("s") * 2 + lax.axis_index("c")
    base = pl.multiple_of(wid * PER_W, PER_W)
    pltpu.sync_copy(idx_hbm.at[pl.ds(base, PER_W)], idx_v)
    pltpu.sync_copy(pe_hbm, pe_v)

    wbufs, gsems, wsems = (w0, w1), (g0, g1), (wr0, wr1)

    def issue_gather(it, b):
        off = pl.multiple_of(it * SEQ, 8)
        pltpu.async_copy(vocab2_hbm.at[idx_v.at[pl.ds(off, GHALF)]],
                         wbufs[b].at[pl.ds(0, GHALF)], gsems[b])
        pltpu.async_copy(vocab2_hbm.at[idx_v.at[pl.ds(off + GHALF, SEQ - GHALF)]],
                         wbufs[b].at[pl.ds(GHALF, SEQ - GHALF)], gsems[b])

    def wait_gather(b):
        pltpu.make_async_copy(vocab2_hbm.at[pl.ds(0, SEQ)], wbufs[b],
                              gsems[b]).wait()

    def issue_write(it, b):
        off = pl.multiple_of(it * SEQ, 8)
        pltpu.async_copy(obuf, out_hbm.at[pl.ds(base + off, SEQ)], wsems[b])

    def wait_write(b):
        pltpu.make_async_copy(obuf, out_hbm.at[pl.ds(0, SEQ)],
                              wsems[b]).wait()

    def compute(b):
        wbuf = wbufs[b]

        def row_body(r, c2):
            for u in range(UNROLL):
                i = r * UNROLL + u
                for j in range(DIM // 16):
                    s = pl.ds(j * 16, 16)
                    obuf[i, s] = wbuf[i, s] + pe_v[i, s]
            return c2

        lax.fori_loop(0, SEQ // UNROLL, row_body, 0)

    issue_gather(0, 0)

    def pair_body(gi, carry):
        for b in range(2):
            it = gi * 2 + b
            nb = 1 - b
            if b == 0:
                @pl.when(gi >= 1)
                def _():
                    wait_write(nb)
                issue_gather(it + 1, nb)
            else:
                wait_write(nb)

                @pl.when(gi + 1 < ROWS_W // 2)
                def _():
                    issue_gather(it + 1, nb)
            wait_gather(b)
            compute(b)
            issue_write(it, b)
        return carry

    lax.fori_loop(0, ROWS_W // 2, pair_body, 0)
    wait_write(1)


def kernel(tokens, token_types, word_table, type_table):
    idx = (tokens.astype(jnp.int32)
           + VOCAB * token_types.astype(jnp.int32)).reshape(NUM_TOK)
    pe = _positional_encoding(SEQ, DIM)
    word2 = word_table.reshape(VOCAB // 2, PDIM)
    vocab2 = _prep_kernel(word2, type_table)
    out = _emb_kernel(idx, pe, vocab2)
    return out.reshape(BATCH, SEQ, DIM)
